# baseline (speedup 1.0000x reference)
import functools

import numpy as np
import jax
import jax.numpy as jnp
from jax import lax
from jax.experimental import pallas as pl
from jax.experimental.pallas import tpu as pltpu
from jax.experimental.pallas import tpu_sc as plsc

T, D, H, DH = 2048, 768, 12, 64
E, C, F = 64, 64, 1536
NSLOT = E * C
V = 32000

_NC, _NS, _L = 2, 16, 16
_NW = _NC * _NS

_F32 = jnp.float32


def _pe_table():
    pos = np.arange(T)[:, None].astype(np.float64)
    i = np.arange(D // 2)[None, :].astype(np.float64)
    ang = pos / np.power(10000.0, 2.0 * i / D)
    pe = np.zeros((T, D), dtype=np.float32)
    pe[:, 0::2] = np.sin(ang)
    pe[:, 1::2] = np.cos(ang)
    return pe


_PE = _pe_table()


def _sc_mesh():
    return plsc.VectorSubcoreMesh(
        core_axis_name="c", subcore_axis_name="s",
        num_cores=_NC, num_subcores=_NS)


def _sc_row_gather(table, idx, nrows):
    bpw = nrows // _NW
    d = table.shape[1]

    @functools.partial(
        pl.kernel, mesh=_sc_mesh(),
        compiler_params=pltpu.CompilerParams(needs_layout_passes=False),
        out_type=jax.ShapeDtypeStruct((nrows, d), _F32),
        scratch_types=[
            pltpu.VMEM((bpw,), jnp.int32),
            pltpu.VMEM((bpw, d), _F32),
            pltpu.SemaphoreType.DMA,
        ],
        name=f"sc_gather_{nrows}x{d}",
    )
    def k(table_hbm, idx_hbm, out_hbm, idx_v, rows_v, sem):
        wid = lax.axis_index("s") * _NC + lax.axis_index("c")
        base = wid * bpw
        pltpu.sync_copy(idx_hbm.at[pl.ds(base, bpw)], idx_v)
        pltpu.async_copy(table_hbm.at[idx_v], rows_v, sem).wait()
        pltpu.sync_copy(rows_v, out_hbm.at[pl.ds(base, bpw)])

    return k(table, idx)


def _sc_inv_scatter(slot):
    @functools.partial(
        pl.kernel, mesh=_sc_mesh(),
        compiler_params=pltpu.CompilerParams(needs_layout_passes=False),
        out_type=jax.ShapeDtypeStruct((NSLOT,), jnp.int32),
        scratch_types=[
            pltpu.VMEM((T,), jnp.int32),
            pltpu.VMEM((NSLOT,), jnp.int32),
        ],
        name="sc_inv_scatter",
    )
    def k(slot_hbm, out_hbm, slots_v, inv_v):
        wid = lax.axis_index("s") * _NC + lax.axis_index("c")

        @pl.when(wid == 0)
        def _():
            pltpu.sync_copy(slot_hbm, slots_v)

            def init(j, _):
                inv_v[pl.ds(j * _L, _L)] = (
                    (lax.iota(jnp.int32, 16) + j * _L) & (T - 1))
                return 0

            lax.fori_loop(0, NSLOT // _L, init, 0)

            def scat(i, _):
                sv = slots_v[pl.ds(i * _L, _L)]
                tv = lax.iota(jnp.int32, 16) + i * _L
                msk = sv < NSLOT
                plsc.store_scatter(
                    inv_v, [jnp.minimum(sv, NSLOT - 1)], tv, mask=msk)
                return 0

            lax.fori_loop(0, T // _L, scat, 0)
            pltpu.sync_copy(inv_v, out_hbm)

    return k(slot)


_BF = jnp.bfloat16


def _qkv_body(x0, pe, wq, bq, wk, bk, wv, bv, xo, qo, ko, vo):
    x = x0[...] + pe[...]
    xo[...] = x
    xb = x.astype(_BF)
    cT = (((0,), (1,)), ((), ()))
    qo[...] = ((lax.dot_general(wq[...].astype(_BF), xb, cT,
                                preferred_element_type=_F32)
                + bq[...]) * 0.125).astype(_BF)
    ko[...] = (lax.dot_general(wk[...].astype(_BF), xb, cT,
                               preferred_element_type=_F32)
               + bk[...]).astype(_BF)
    vo[...] = (lax.dot_general(wv[...].astype(_BF), xb, cT,
                               preferred_element_type=_F32)
               + bv[...]).astype(_BF)


def _qkv(x0, pe, wq, bq, wk, bk, wv, bv):
    return pl.pallas_call(
        _qkv_body,
        out_shape=[jax.ShapeDtypeStruct((T, D), _F32)]
        + [jax.ShapeDtypeStruct((D, T), _BF)] * 3,
    )(x0, pe, wq, bq, wk, bk, wv, bv)


def _attn_router_body(x, q, k, v, wo, bo, ln1s, ln1b, wr,
                      yo, slot_o, slotc_o, gate_o, z_o, acc):
    h = pl.program_id(0)

    @pl.when(h < H)
    def _():
        s = lax.dot_general(q[...], k[...], (((0,), (0,)), ((), ())),
                            preferred_element_type=_F32)
        e = jnp.exp(s)
        denom = jnp.sum(e, axis=-1, keepdims=True)
        oh = lax.dot_general(e.astype(_BF), v[...], (((1,), (1,)), ((), ())),
                             preferred_element_type=_F32) / denom
        contrib = jnp.dot(oh.astype(_BF), wo[...].astype(_BF),
                          preferred_element_type=_F32)

        @pl.when(h == 0)
        def _():
            acc[...] = contrib

        @pl.when(h != 0)
        def _():
            acc[...] += contrib

    @pl.when(h == H)
    def _():
        y = x[...] + acc[...] + bo[...]
        m = jnp.mean(y, axis=-1, keepdims=True)
        var = jnp.mean((y - m) ** 2, axis=-1, keepdims=True)
        y = (y - m) / jnp.sqrt(var + 1e-5) * ln1s[...] + ln1b[...]
        yo[...] = y
        logits = jnp.dot(y, wr[...], preferred_element_type=_F32)
        mx = jnp.max(logits, axis=-1, keepdims=True)
        ex = jnp.exp(logits - mx)
        se = jnp.sum(ex, axis=-1, keepdims=True)
        probs = ex / se
        pm = jnp.max(probs, axis=-1, keepdims=True)
        lane = lax.broadcasted_iota(jnp.int32, (T, E), 1)
        eidx = jnp.min(jnp.where(probs == pm, lane, E), axis=-1,
                       keepdims=True)
        mask = (lane == eidx).astype(_F32)
        tri = (lax.broadcasted_iota(jnp.int32, (T, T), 0)
               >= lax.broadcasted_iota(jnp.int32, (T, T), 1)).astype(_F32)
        cs = lax.dot_general(tri, mask, (((1,), (0,)), ((), ())),
                             preferred_element_type=_F32)
        posi = jnp.sum(cs * mask, axis=-1, keepdims=True) - 1.0
        keep = posi < C
        slot = eidx * C + posi.astype(jnp.int32)
        slot_o[...] = jnp.where(keep, slot, NSLOT)
        tok = lax.broadcasted_iota(jnp.int32, (T, 1), 0)
        slotc_o[...] = jnp.where(keep, slot, tok)
        gate_o[...] = jnp.where(keep, pm, 0.0)
        lse = mx + jnp.log(se)
        z_o[...] = jnp.mean(lse ** 2, keepdims=True)


def _attn_router(x, q, k, v, wo, bo, ln1s, ln1b, wr):
    hcl = lambda h: (jnp.minimum(h, H - 1), 0)
    full = lambda h: (0, 0)
    return pl.pallas_call(
        _attn_router_body,
        grid=(H + 1,),
        in_specs=[
            pl.BlockSpec((T, D), full),
            pl.BlockSpec((DH, T), hcl),
            pl.BlockSpec((DH, T), hcl),
            pl.BlockSpec((DH, T), hcl),
            pl.BlockSpec((DH, D), hcl),
            pl.BlockSpec((1, D), full),
            pl.BlockSpec((1, D), full),
            pl.BlockSpec((1, D), full),
            pl.BlockSpec((D, E), full),
        ],
        out_specs=[
            pl.BlockSpec((T, D), full),
            pl.BlockSpec((T, 1), full),
            pl.BlockSpec((T, 1), full),
            pl.BlockSpec((T, 1), full),
            pl.BlockSpec((1, 1), full),
        ],
        out_shape=[
            jax.ShapeDtypeStruct((T, D), _F32),
            jax.ShapeDtypeStruct((T, 1), jnp.int32),
            jax.ShapeDtypeStruct((T, 1), jnp.int32),
            jax.ShapeDtypeStruct((T, 1), _F32),
            jax.ShapeDtypeStruct((1, 1), _F32),
        ],
        scratch_shapes=[pltpu.VMEM((T, D), _F32)],
        compiler_params=pltpu.CompilerParams(
            dimension_semantics=("arbitrary",),
            vmem_limit_bytes=120 * 1024 * 1024,
            fuse_transposed_lhs_in_matmul=True),
    )(x, q, k, v, wo, bo, ln1s, ln1b, wr)


def _expert_body(xin, w1, b1, w2, b2, res, out):
    del res
    hh = jax.nn.gelu(
        jnp.dot(xin[...], w1[0], preferred_element_type=_F32) + b1[0])
    out[...] = jnp.dot(hh, w2[0], preferred_element_type=_F32) + b2[0]


def _experts(exp_in, ew1, eb1, ew2, eb2, res):
    return pl.pallas_call(
        _expert_body,
        grid=(E,),
        in_specs=[
            pl.BlockSpec((C, D), lambda e: (e, 0)),
            pl.BlockSpec((1, D, F), lambda e: (e, 0, 0)),
            pl.BlockSpec((1, 1, F), lambda e: (e, 0, 0)),
            pl.BlockSpec((1, F, D), lambda e: (e, 0, 0)),
            pl.BlockSpec((1, 1, D), lambda e: (e, 0, 0)),
            pl.BlockSpec((8, 128), lambda e: (0, 0)),
        ],
        out_specs=pl.BlockSpec((C, D), lambda e: (e, 0)),
        out_shape=jax.ShapeDtypeStruct((NSLOT, D), _F32),
        compiler_params=pltpu.CompilerParams(
            dimension_semantics=("arbitrary",)),
    )(exp_in, ew1, eb1, ew2, eb2, res)


def _final_body(y, comb, gate, res, ln2s, ln2b, flns, flnb, out):
    z = y[...] + gate[...] * comb[...] + res[...]
    m2 = jnp.mean(z, axis=-1, keepdims=True)
    v2 = jnp.mean((z - m2) ** 2, axis=-1, keepdims=True)
    z = (z - m2) / jnp.sqrt(v2 + 1e-5) * ln2s[...] + ln2b[...]
    m3 = jnp.mean(z, axis=-1, keepdims=True)
    v3 = jnp.mean((z - m3) ** 2, axis=-1, keepdims=True)
    out[...] = (z - m3) / jnp.sqrt(v3 + 1e-5) * flns[...] + flnb[...]


def _final(y, comb, gate, res, ln2s, ln2b, flns, flnb):
    return pl.pallas_call(
        _final_body,
        out_shape=jax.ShapeDtypeStruct((T, D), _F32),
    )(y, comb, gate, res, ln2s, ln2b, flns, flnb)


def _resffn_body(y, rw1, rb1, rw2, rb2, res_o):
    hh = jax.nn.gelu(
        jnp.dot(y[...].astype(_BF), rw1[...].astype(_BF),
                preferred_element_type=_F32) + rb1[...])
    res_o[...] = jnp.dot(hh.astype(_BF), rw2[...].astype(_BF),
                         preferred_element_type=_F32) + rb2[...]


def _resffn(y, rw1, rb1, rw2, rb2):
    return pl.pallas_call(
        _resffn_body,
        out_shape=jax.ShapeDtypeStruct((T, D), _F32),
    )(y, rw1, rb1, rw2, rb2)


def kernel(input_ids, emb_table, wq, bq, wk, bk, wv, bv, wo, bo, ln1_s, ln1_b,
           wr, ew1, eb1, ew2, eb2, rw1, rb1, rw2, rb2, ln2_s, ln2_b,
           fln_s, fln_b):
    ids = input_ids.reshape(T)
    x0 = _sc_row_gather(emb_table, ids, T)
    pe = jnp.asarray(_PE)
    r2 = lambda a: a.reshape(1, -1)
    rc = lambda a: a.reshape(-1, 1)
    x, q, k, v = _qkv(x0, pe, wq, rc(bq), wk, rc(bk), wv, rc(bv))
    y, slot2, slotc2, gate2, z = _attn_router(x, q, k, v, wo, r2(bo),
                                              r2(ln1_s), r2(ln1_b), wr)
    res = _resffn(y, rw1, r2(rb1), rw2, r2(rb2))
    inv = _sc_inv_scatter(slot2.reshape(T))
    exp_in = _sc_row_gather(y, inv, NSLOT)
    exp_out = _experts(exp_in, ew1, eb1.reshape(E, 1, F), ew2,
                       eb2.reshape(E, 1, D), res)
    comb = _sc_row_gather(exp_out, slotc2.reshape(T), T)
    out = _final(y, comb, gate2, res, r2(ln2_s), r2(ln2_b),
                 r2(fln_s), r2(fln_b))
    return out.reshape(1, T, D), z[0, 0]


# speedup vs baseline: 1.0002x; 1.0002x over previous
import functools

import numpy as np
import jax
import jax.numpy as jnp
from jax import lax
from jax.experimental import pallas as pl
from jax.experimental.pallas import tpu as pltpu
from jax.experimental.pallas import tpu_sc as plsc

T, D, H, DH = 2048, 768, 12, 64
E, C, F = 64, 64, 1536
NSLOT = E * C
V = 32000

_NC, _NS, _L = 2, 16, 16
_NW = _NC * _NS

_F32 = jnp.float32


def _pe_table():
    pos = np.arange(T)[:, None].astype(np.float64)
    i = np.arange(D // 2)[None, :].astype(np.float64)
    ang = pos / np.power(10000.0, 2.0 * i / D)
    pe = np.zeros((T, D), dtype=np.float32)
    pe[:, 0::2] = np.sin(ang)
    pe[:, 1::2] = np.cos(ang)
    return pe


_PE = _pe_table()


def _sc_mesh():
    return plsc.VectorSubcoreMesh(
        core_axis_name="c", subcore_axis_name="s",
        num_cores=_NC, num_subcores=_NS)


def _sc_row_gather(table, idx, nrows):
    bpw = nrows // _NW
    d = table.shape[1]

    @functools.partial(
        pl.kernel, mesh=_sc_mesh(),
        compiler_params=pltpu.CompilerParams(needs_layout_passes=False),
        out_type=jax.ShapeDtypeStruct((nrows, d), _F32),
        scratch_types=[
            pltpu.VMEM((bpw,), jnp.int32),
            pltpu.VMEM((bpw, d), _F32),
            pltpu.SemaphoreType.DMA,
        ],
        name=f"sc_gather_{nrows}x{d}",
    )
    def k(table_hbm, idx_hbm, out_hbm, idx_v, rows_v, sem):
        wid = lax.axis_index("s") * _NC + lax.axis_index("c")
        base = wid * bpw
        pltpu.sync_copy(idx_hbm.at[pl.ds(base, bpw)], idx_v)
        pltpu.async_copy(table_hbm.at[idx_v], rows_v, sem).wait()
        pltpu.sync_copy(rows_v, out_hbm.at[pl.ds(base, bpw)])

    return k(table, idx)


def _sc_inv_scatter(slot):
    @functools.partial(
        pl.kernel, mesh=_sc_mesh(),
        compiler_params=pltpu.CompilerParams(needs_layout_passes=False),
        out_type=jax.ShapeDtypeStruct((NSLOT,), jnp.int32),
        scratch_types=[
            pltpu.VMEM((T,), jnp.int32),
            pltpu.VMEM((NSLOT,), jnp.int32),
        ],
        name="sc_inv_scatter",
    )
    def k(slot_hbm, out_hbm, slots_v, inv_v):
        wid = lax.axis_index("s") * _NC + lax.axis_index("c")

        @pl.when(wid == 0)
        def _():
            pltpu.sync_copy(slot_hbm, slots_v)

            def init(j, _):
                inv_v[pl.ds(j * _L, _L)] = (
                    (lax.iota(jnp.int32, 16) + j * _L) & (T - 1))
                return 0

            lax.fori_loop(0, NSLOT // _L, init, 0)

            def scat(i, _):
                sv = slots_v[pl.ds(i * _L, _L)]
                tv = lax.iota(jnp.int32, 16) + i * _L
                msk = sv < NSLOT
                plsc.store_scatter(
                    inv_v, [jnp.minimum(sv, NSLOT - 1)], tv, mask=msk)
                return 0

            lax.fori_loop(0, T // _L, scat, 0)
            pltpu.sync_copy(inv_v, out_hbm)

    return k(slot)


_BF = jnp.bfloat16


def _qkv_body(x0, pe, wq, bq, wk, bk, wv, bv, xo, qo, ko, vo):
    x = x0[...] + pe[...]
    xo[...] = x
    xb = x.astype(_BF)
    cT = (((0,), (1,)), ((), ()))
    qo[...] = ((lax.dot_general(wq[...].astype(_BF), xb, cT,
                                preferred_element_type=_F32)
                + bq[...]) * 0.125).astype(_BF)
    ko[...] = (lax.dot_general(wk[...].astype(_BF), xb, cT,
                               preferred_element_type=_F32)
               + bk[...]).astype(_BF)
    vo[...] = (lax.dot_general(wv[...].astype(_BF), xb, cT,
                               preferred_element_type=_F32)
               + bv[...]).astype(_BF)


def _qkv(x0, pe, wq, bq, wk, bk, wv, bv):
    return pl.pallas_call(
        _qkv_body,
        out_shape=[jax.ShapeDtypeStruct((T, D), _F32)]
        + [jax.ShapeDtypeStruct((D, T), _BF)] * 3,
    )(x0, pe, wq, bq, wk, bk, wv, bv)


def _attn_router_body(x, q, k, v, wo, bo, ln1s, ln1b, wr,
                      yo, slot_o, slotc_o, gate_o, z_o, acc):
    h = pl.program_id(0)

    @pl.when(h < H)
    def _():
        s = lax.dot_general(q[...], k[...], (((0,), (0,)), ((), ())),
                            preferred_element_type=_F32)
        e = jnp.exp(s)
        denom = jnp.sum(e, axis=-1, keepdims=True)
        oh = lax.dot_general(e.astype(_BF), v[...], (((1,), (1,)), ((), ())),
                             preferred_element_type=_F32) / denom
        contrib = jnp.dot(oh.astype(_BF), wo[...].astype(_BF),
                          preferred_element_type=_F32)

        @pl.when(h == 0)
        def _():
            acc[...] = contrib

        @pl.when(h != 0)
        def _():
            acc[...] += contrib

    @pl.when(h == H)
    def _():
        y = x[...] + acc[...] + bo[...]
        m = jnp.mean(y, axis=-1, keepdims=True)
        var = jnp.mean((y - m) ** 2, axis=-1, keepdims=True)
        y = (y - m) / jnp.sqrt(var + 1e-5) * ln1s[...] + ln1b[...]
        yo[...] = y
        logits = jnp.dot(y, wr[...], preferred_element_type=_F32)
        mx = jnp.max(logits, axis=-1, keepdims=True)
        ex = jnp.exp(logits - mx)
        se = jnp.sum(ex, axis=-1, keepdims=True)
        probs = ex / se
        pm = jnp.max(probs, axis=-1, keepdims=True)
        lane = lax.broadcasted_iota(jnp.int32, (T, E), 1)
        eidx = jnp.min(jnp.where(probs == pm, lane, E), axis=-1,
                       keepdims=True)
        mask = (lane == eidx).astype(_F32)
        tri = (lax.broadcasted_iota(jnp.int32, (T, T), 0)
               >= lax.broadcasted_iota(jnp.int32, (T, T), 1)).astype(_F32)
        cs = lax.dot_general(tri, mask, (((1,), (0,)), ((), ())),
                             preferred_element_type=_F32)
        posi = jnp.sum(cs * mask, axis=-1, keepdims=True) - 1.0
        keep = posi < C
        slot = eidx * C + posi.astype(jnp.int32)
        slot_o[...] = jnp.where(keep, slot, NSLOT)
        tok = lax.broadcasted_iota(jnp.int32, (T, 1), 0)
        slotc_o[...] = jnp.where(keep, slot, tok)
        gate_o[...] = jnp.where(keep, pm, 0.0)
        lse = mx + jnp.log(se)
        z_o[...] = jnp.mean(lse ** 2, keepdims=True)


def _attn_router(x, q, k, v, wo, bo, ln1s, ln1b, wr):
    hcl = lambda h: (jnp.minimum(h, H - 1), 0)
    full = lambda h: (0, 0)
    return pl.pallas_call(
        _attn_router_body,
        grid=(H + 1,),
        in_specs=[
            pl.BlockSpec((T, D), full),
            pl.BlockSpec((DH, T), hcl),
            pl.BlockSpec((DH, T), hcl),
            pl.BlockSpec((DH, T), hcl),
            pl.BlockSpec((DH, D), hcl),
            pl.BlockSpec((1, D), full),
            pl.BlockSpec((1, D), full),
            pl.BlockSpec((1, D), full),
            pl.BlockSpec((D, E), full),
        ],
        out_specs=[
            pl.BlockSpec((T, D), full),
            pl.BlockSpec((T, 1), full),
            pl.BlockSpec((T, 1), full),
            pl.BlockSpec((T, 1), full),
            pl.BlockSpec((1, 1), full),
        ],
        out_shape=[
            jax.ShapeDtypeStruct((T, D), _F32),
            jax.ShapeDtypeStruct((T, 1), jnp.int32),
            jax.ShapeDtypeStruct((T, 1), jnp.int32),
            jax.ShapeDtypeStruct((T, 1), _F32),
            jax.ShapeDtypeStruct((1, 1), _F32),
        ],
        scratch_shapes=[pltpu.VMEM((T, D), _F32)],
        compiler_params=pltpu.CompilerParams(
            dimension_semantics=("arbitrary",),
            vmem_limit_bytes=120 * 1024 * 1024),
    )(x, q, k, v, wo, bo, ln1s, ln1b, wr)


def _expert_body(xin, w1, b1, w2, b2, res, out):
    del res
    hh = jax.nn.gelu(
        jnp.dot(xin[...], w1[0], preferred_element_type=_F32) + b1[0])
    out[...] = jnp.dot(hh, w2[0], preferred_element_type=_F32) + b2[0]


def _experts(exp_in, ew1, eb1, ew2, eb2, res):
    return pl.pallas_call(
        _expert_body,
        grid=(E,),
        in_specs=[
            pl.BlockSpec((C, D), lambda e: (e, 0)),
            pl.BlockSpec((1, D, F), lambda e: (e, 0, 0)),
            pl.BlockSpec((1, 1, F), lambda e: (e, 0, 0)),
            pl.BlockSpec((1, F, D), lambda e: (e, 0, 0)),
            pl.BlockSpec((1, 1, D), lambda e: (e, 0, 0)),
            pl.BlockSpec((8, 128), lambda e: (0, 0)),
        ],
        out_specs=pl.BlockSpec((C, D), lambda e: (e, 0)),
        out_shape=jax.ShapeDtypeStruct((NSLOT, D), _F32),
        compiler_params=pltpu.CompilerParams(
            dimension_semantics=("arbitrary",)),
    )(exp_in, ew1, eb1, ew2, eb2, res)


def _final_body(y, comb, gate, res, ln2s, ln2b, flns, flnb, out):
    z = y[...] + gate[...] * comb[...] + res[...]
    m2 = jnp.mean(z, axis=-1, keepdims=True)
    v2 = jnp.mean((z - m2) ** 2, axis=-1, keepdims=True)
    z = (z - m2) / jnp.sqrt(v2 + 1e-5) * ln2s[...] + ln2b[...]
    m3 = jnp.mean(z, axis=-1, keepdims=True)
    v3 = jnp.mean((z - m3) ** 2, axis=-1, keepdims=True)
    out[...] = (z - m3) / jnp.sqrt(v3 + 1e-5) * flns[...] + flnb[...]


def _final(y, comb, gate, res, ln2s, ln2b, flns, flnb):
    return pl.pallas_call(
        _final_body,
        out_shape=jax.ShapeDtypeStruct((T, D), _F32),
    )(y, comb, gate, res, ln2s, ln2b, flns, flnb)


def _resffn_body(y, rw1, rb1, rw2, rb2, res_o):
    hh = jax.nn.gelu(
        jnp.dot(y[...].astype(_BF), rw1[...].astype(_BF),
                preferred_element_type=_F32) + rb1[...])
    res_o[...] = jnp.dot(hh.astype(_BF), rw2[...].astype(_BF),
                         preferred_element_type=_F32) + rb2[...]


def _resffn(y, rw1, rb1, rw2, rb2):
    return pl.pallas_call(
        _resffn_body,
        out_shape=jax.ShapeDtypeStruct((T, D), _F32),
    )(y, rw1, rb1, rw2, rb2)


def kernel(input_ids, emb_table, wq, bq, wk, bk, wv, bv, wo, bo, ln1_s, ln1_b,
           wr, ew1, eb1, ew2, eb2, rw1, rb1, rw2, rb2, ln2_s, ln2_b,
           fln_s, fln_b):
    ids = input_ids.reshape(T)
    x0 = _sc_row_gather(emb_table, ids, T)
    pe = jnp.asarray(_PE)
    r2 = lambda a: a.reshape(1, -1)
    rc = lambda a: a.reshape(-1, 1)
    x, q, k, v = _qkv(x0, pe, wq, rc(bq), wk, rc(bk), wv, rc(bv))
    y, slot2, slotc2, gate2, z = _attn_router(x, q, k, v, wo, r2(bo),
                                              r2(ln1_s), r2(ln1_b), wr)
    res = _resffn(y, rw1, r2(rb1), rw2, r2(rb2))
    inv = _sc_inv_scatter(slot2.reshape(T))
    exp_in = _sc_row_gather(y, inv, NSLOT)
    exp_out = _experts(exp_in, ew1, eb1.reshape(E, 1, F), ew2,
                       eb2.reshape(E, 1, D), res)
    comb = _sc_row_gather(exp_out, slotc2.reshape(T), T)
    out = _final(y, comb, gate2, res, r2(ln2_s), r2(ln2_b),
                 r2(fln_s), r2(fln_b))
    return out.reshape(1, T, D), z[0, 0]


# speedup vs baseline: 1.0011x; 1.0009x over previous
import functools

import numpy as np
import jax
import jax.numpy as jnp
from jax import lax
from jax.experimental import pallas as pl
from jax.experimental.pallas import tpu as pltpu
from jax.experimental.pallas import tpu_sc as plsc

T, D, H, DH = 2048, 768, 12, 64
E, C, F = 64, 64, 1536
NSLOT = E * C
V = 32000

_NC, _NS, _L = 2, 16, 16
_NW = _NC * _NS

_F32 = jnp.float32


def _pe_table():
    pos = np.arange(T)[:, None].astype(np.float64)
    i = np.arange(D // 2)[None, :].astype(np.float64)
    ang = pos / np.power(10000.0, 2.0 * i / D)
    pe = np.zeros((T, D), dtype=np.float32)
    pe[:, 0::2] = np.sin(ang)
    pe[:, 1::2] = np.cos(ang)
    return pe


_PE = _pe_table()


def _sc_mesh():
    return plsc.VectorSubcoreMesh(
        core_axis_name="c", subcore_axis_name="s",
        num_cores=_NC, num_subcores=_NS)


def _sc_row_gather(table, idx, nrows):
    bpw = nrows // _NW
    d = table.shape[1]

    @functools.partial(
        pl.kernel, mesh=_sc_mesh(),
        compiler_params=pltpu.CompilerParams(needs_layout_passes=False),
        out_type=jax.ShapeDtypeStruct((nrows, d), _F32),
        scratch_types=[
            pltpu.VMEM((bpw,), jnp.int32),
            pltpu.VMEM((bpw, d), _F32),
            pltpu.SemaphoreType.DMA,
        ],
        name=f"sc_gather_{nrows}x{d}",
    )
    def k(table_hbm, idx_hbm, out_hbm, idx_v, rows_v, sem):
        wid = lax.axis_index("s") * _NC + lax.axis_index("c")
        base = wid * bpw
        pltpu.sync_copy(idx_hbm.at[pl.ds(base, bpw)], idx_v)
        pltpu.async_copy(table_hbm.at[idx_v], rows_v, sem).wait()
        pltpu.sync_copy(rows_v, out_hbm.at[pl.ds(base, bpw)])

    return k(table, idx)


def _sc_inv_scatter(slot):
    @functools.partial(
        pl.kernel, mesh=_sc_mesh(),
        compiler_params=pltpu.CompilerParams(needs_layout_passes=False),
        out_type=jax.ShapeDtypeStruct((NSLOT,), jnp.int32),
        scratch_types=[
            pltpu.VMEM((T,), jnp.int32),
            pltpu.VMEM((NSLOT,), jnp.int32),
        ],
        name="sc_inv_scatter",
    )
    def k(slot_hbm, out_hbm, slots_v, inv_v):
        wid = lax.axis_index("s") * _NC + lax.axis_index("c")

        @pl.when(wid == 0)
        def _():
            pltpu.sync_copy(slot_hbm, slots_v)

            def init(j, _):
                inv_v[pl.ds(j * _L, _L)] = (
                    (lax.iota(jnp.int32, 16) + j * _L) & (T - 1))
                return 0

            lax.fori_loop(0, NSLOT // _L, init, 0)

            def scat(i, _):
                sv = slots_v[pl.ds(i * _L, _L)]
                tv = lax.iota(jnp.int32, 16) + i * _L
                msk = sv < NSLOT
                plsc.store_scatter(
                    inv_v, [jnp.minimum(sv, NSLOT - 1)], tv, mask=msk)
                return 0

            lax.fori_loop(0, T // _L, scat, 0)
            pltpu.sync_copy(inv_v, out_hbm)

    return k(slot)


_BF = jnp.bfloat16


def _qkv_body(x0, pe, wq, bq, wk, bk, wv, bv, xo, qo, ko, vo):
    x = x0[...] + pe[...]
    xo[...] = x
    xb = x.astype(_BF)
    cT = (((0,), (1,)), ((), ()))
    qo[...] = ((lax.dot_general(wq[...].astype(_BF), xb, cT,
                                preferred_element_type=_F32)
                + bq[...]) * 0.125).astype(_BF)
    ko[...] = (lax.dot_general(wk[...].astype(_BF), xb, cT,
                               preferred_element_type=_F32)
               + bk[...]).astype(_BF)
    vo[...] = (lax.dot_general(wv[...].astype(_BF), xb, cT,
                               preferred_element_type=_F32)
               + bv[...]).astype(_BF)


def _qkv(x0, pe, wq, bq, wk, bk, wv, bv):
    return pl.pallas_call(
        _qkv_body,
        out_shape=[jax.ShapeDtypeStruct((T, D), _F32)]
        + [jax.ShapeDtypeStruct((D, T), _BF)] * 3,
    )(x0, pe, wq, bq, wk, bk, wv, bv)


def _attn_router_body(x, q, k, v, wo, bo, ln1s, ln1b, wr,
                      yo, slot_o, slotc_o, gate_o, z_o, acc):
    h = pl.program_id(0)

    @pl.when(h < H)
    def _():
        s = lax.dot_general(q[...], k[...], (((0,), (0,)), ((), ())),
                            preferred_element_type=_F32)
        e = jnp.exp(s)
        denom = jnp.sum(e, axis=-1, keepdims=True)
        oh = lax.dot_general(e.astype(_BF), v[...], (((1,), (1,)), ((), ())),
                             preferred_element_type=_F32) / denom
        contrib = jnp.dot(oh.astype(_BF), wo[...].astype(_BF),
                          preferred_element_type=_F32)

        @pl.when(h == 0)
        def _():
            acc[...] = contrib

        @pl.when(h != 0)
        def _():
            acc[...] += contrib

    @pl.when(h == H)
    def _():
        y = x[...] + acc[...] + bo[...]
        m = jnp.mean(y, axis=-1, keepdims=True)
        var = jnp.mean((y - m) ** 2, axis=-1, keepdims=True)
        y = (y - m) / jnp.sqrt(var + 1e-5) * ln1s[...] + ln1b[...]
        yo[...] = y
        logits = jnp.dot(y, wr[...], preferred_element_type=_F32)
        mx = jnp.max(logits, axis=-1, keepdims=True)
        ex = jnp.exp(logits - mx)
        se = jnp.sum(ex, axis=-1, keepdims=True)
        probs = ex / se
        pm = jnp.max(probs, axis=-1, keepdims=True)
        lane = lax.broadcasted_iota(jnp.int32, (T, E), 1)
        eidx = jnp.min(jnp.where(probs == pm, lane, E), axis=-1,
                       keepdims=True)
        mask = (lane == eidx).astype(_F32)
        tri = (lax.broadcasted_iota(jnp.int32, (T, T), 0)
               >= lax.broadcasted_iota(jnp.int32, (T, T), 1)).astype(_F32)
        cs = lax.dot_general(tri.astype(_BF), mask.astype(_BF),
                             (((1,), (0,)), ((), ())),
                             preferred_element_type=_F32)
        posi = jnp.sum(cs * mask, axis=-1, keepdims=True) - 1.0
        keep = posi < C
        slot = eidx * C + posi.astype(jnp.int32)
        slot_o[...] = jnp.where(keep, slot, NSLOT)
        tok = lax.broadcasted_iota(jnp.int32, (T, 1), 0)
        slotc_o[...] = jnp.where(keep, slot, tok)
        gate_o[...] = jnp.where(keep, pm, 0.0)
        lse = mx + jnp.log(se)
        z_o[...] = jnp.mean(lse ** 2, keepdims=True)


def _attn_router(x, q, k, v, wo, bo, ln1s, ln1b, wr):
    hcl = lambda h: (jnp.minimum(h, H - 1), 0)
    full = lambda h: (0, 0)
    return pl.pallas_call(
        _attn_router_body,
        grid=(H + 1,),
        in_specs=[
            pl.BlockSpec((T, D), full),
            pl.BlockSpec((DH, T), hcl),
            pl.BlockSpec((DH, T), hcl),
            pl.BlockSpec((DH, T), hcl),
            pl.BlockSpec((DH, D), hcl),
            pl.BlockSpec((1, D), full),
            pl.BlockSpec((1, D), full),
            pl.BlockSpec((1, D), full),
            pl.BlockSpec((D, E), full),
        ],
        out_specs=[
            pl.BlockSpec((T, D), full),
            pl.BlockSpec((T, 1), full),
            pl.BlockSpec((T, 1), full),
            pl.BlockSpec((T, 1), full),
            pl.BlockSpec((1, 1), full),
        ],
        out_shape=[
            jax.ShapeDtypeStruct((T, D), _F32),
            jax.ShapeDtypeStruct((T, 1), jnp.int32),
            jax.ShapeDtypeStruct((T, 1), jnp.int32),
            jax.ShapeDtypeStruct((T, 1), _F32),
            jax.ShapeDtypeStruct((1, 1), _F32),
        ],
        scratch_shapes=[pltpu.VMEM((T, D), _F32)],
        compiler_params=pltpu.CompilerParams(
            dimension_semantics=("arbitrary",),
            vmem_limit_bytes=120 * 1024 * 1024),
    )(x, q, k, v, wo, bo, ln1s, ln1b, wr)


def _expert_body(xin, w1, b1, w2, b2, res, out):
    del res
    hh = jax.nn.gelu(
        jnp.dot(xin[...], w1[0], preferred_element_type=_F32) + b1[0])
    out[...] = jnp.dot(hh, w2[0], preferred_element_type=_F32) + b2[0]


def _experts(exp_in, ew1, eb1, ew2, eb2, res):
    return pl.pallas_call(
        _expert_body,
        grid=(E,),
        in_specs=[
            pl.BlockSpec((C, D), lambda e: (e, 0)),
            pl.BlockSpec((1, D, F), lambda e: (e, 0, 0)),
            pl.BlockSpec((1, 1, F), lambda e: (e, 0, 0)),
            pl.BlockSpec((1, F, D), lambda e: (e, 0, 0)),
            pl.BlockSpec((1, 1, D), lambda e: (e, 0, 0)),
            pl.BlockSpec((8, 128), lambda e: (0, 0)),
        ],
        out_specs=pl.BlockSpec((C, D), lambda e: (e, 0)),
        out_shape=jax.ShapeDtypeStruct((NSLOT, D), _F32),
        compiler_params=pltpu.CompilerParams(
            dimension_semantics=("arbitrary",)),
    )(exp_in, ew1, eb1, ew2, eb2, res)


def _final_body(y, comb, gate, res, ln2s, ln2b, flns, flnb, out):
    z = y[...] + gate[...] * comb[...] + res[...]
    m2 = jnp.mean(z, axis=-1, keepdims=True)
    v2 = jnp.mean((z - m2) ** 2, axis=-1, keepdims=True)
    z = (z - m2) / jnp.sqrt(v2 + 1e-5) * ln2s[...] + ln2b[...]
    m3 = jnp.mean(z, axis=-1, keepdims=True)
    v3 = jnp.mean((z - m3) ** 2, axis=-1, keepdims=True)
    out[...] = (z - m3) / jnp.sqrt(v3 + 1e-5) * flns[...] + flnb[...]


def _final(y, comb, gate, res, ln2s, ln2b, flns, flnb):
    return pl.pallas_call(
        _final_body,
        out_shape=jax.ShapeDtypeStruct((T, D), _F32),
    )(y, comb, gate, res, ln2s, ln2b, flns, flnb)


def _resffn_body(y, rw1, rb1, rw2, rb2, res_o):
    hh = jax.nn.gelu(
        jnp.dot(y[...].astype(_BF), rw1[...].astype(_BF),
                preferred_element_type=_F32) + rb1[...])
    res_o[...] = jnp.dot(hh.astype(_BF), rw2[...].astype(_BF),
                         preferred_element_type=_F32) + rb2[...]


def _resffn(y, rw1, rb1, rw2, rb2):
    return pl.pallas_call(
        _resffn_body,
        out_shape=jax.ShapeDtypeStruct((T, D), _F32),
    )(y, rw1, rb1, rw2, rb2)


def kernel(input_ids, emb_table, wq, bq, wk, bk, wv, bv, wo, bo, ln1_s, ln1_b,
           wr, ew1, eb1, ew2, eb2, rw1, rb1, rw2, rb2, ln2_s, ln2_b,
           fln_s, fln_b):
    ids = input_ids.reshape(T)
    x0 = _sc_row_gather(emb_table, ids, T)
    pe = jnp.asarray(_PE)
    r2 = lambda a: a.reshape(1, -1)
    rc = lambda a: a.reshape(-1, 1)
    x, q, k, v = _qkv(x0, pe, wq, rc(bq), wk, rc(bk), wv, rc(bv))
    y, slot2, slotc2, gate2, z = _attn_router(x, q, k, v, wo, r2(bo),
                                              r2(ln1_s), r2(ln1_b), wr)
    res = _resffn(y, rw1, r2(rb1), rw2, r2(rb2))
    inv = _sc_inv_scatter(slot2.reshape(T))
    exp_in = _sc_row_gather(y, inv, NSLOT)
    exp_out = _experts(exp_in, ew1, eb1.reshape(E, 1, F), ew2,
                       eb2.reshape(E, 1, D), res)
    comb = _sc_row_gather(exp_out, slotc2.reshape(T), T)
    out = _final(y, comb, gate2, res, r2(ln2_s), r2(ln2_b),
                 r2(fln_s), r2(fln_b))
    return out.reshape(1, T, D), z[0, 0]
